# baseline (speedup 1.0000x reference)
import functools

import jax
import jax.numpy as jnp
from jax import lax
from jax.experimental import pallas as pl
from jax.experimental.pallas import tpu as pltpu
from jax.experimental.pallas import tpu_sc as plsc

SEQ = 8192
D = 2048
N_SLICES = 32
F = 64

NC = 2
NS = 16
NW = NC * NS
BLOCK = SEQ // NW
CH = 16
NBUF = 2
IC = 8
NWAVE = F // IC
NSLOT = 3


def _body(ids_hbm, bounds_hbm, table_hbm, img_hbm, out_hbm,
          idx_v, bounds_v, buf0, buf1, sp,
          gsem0, gsem1, wsem0, wsem1,
          isem0, isem1, isem2, osem0, osem1, osem2):
    cid = lax.axis_index("c")
    sid = lax.axis_index("s")
    wid = sid * NC + cid
    base = pl.multiple_of(wid * BLOCK, BLOCK)

    bufs = (buf0, buf1)
    gsems = (gsem0, gsem1)
    wsems = (wsem0, wsem1)
    isems = (isem0, isem1, isem2)
    osems = (osem0, osem1, osem2)

    def sp_rows(k):
        return pl.ds(sid * (NSLOT * IC) + (k % NSLOT) * IC, IC)

    def img_in(k):
        d = pltpu.make_async_copy(img_hbm.at[pl.ds(wid * F + k * IC, IC)],
                                  sp.at[sp_rows(k)], isems[k % NSLOT])
        d.start()
        return d

    def img_out(k, dst_row):
        d = pltpu.make_async_copy(sp.at[sp_rows(k)],
                                  out_hbm.at[pl.ds(dst_row + k * IC, IC)],
                                  osems[k % NSLOT])
        d.start()
        return d

    img_ds = [None] * NWAVE
    for _k in range(NSLOT):
        img_ds[_k] = img_in(_k)

    pltpu.sync_copy(ids_hbm.at[pl.ds(base, BLOCK)], idx_v)
    pltpu.sync_copy(bounds_hbm, bounds_v.at[pl.ds(0, 2 * N_SLICES)])

    sv = bounds_v[pl.ds(2 * wid, 16)]
    s_start = pl.multiple_of(sv[0], 8)

    def run_gather(pos_off, ch, nch, hook=None):
        def g_start(c):
            b = c % NBUF
            idx_ref = idx_v.at[pl.ds(pos_off + c * ch, ch)]
            d = pltpu.make_async_copy(table_hbm.at[idx_ref],
                                      bufs[b].at[pl.ds(0, ch)], gsems[b])
            d.start()
            return d

        def w_start(c):
            b = c % NBUF
            row0 = base + pos_off + c * ch
            d = pltpu.make_async_copy(bufs[b].at[pl.ds(0, ch)],
                                      out_hbm.at[pl.ds(row0, ch)], wsems[b])
            d.start()
            return d

        gds = [None] * nch
        wds = [None] * nch
        for c in range(min(NBUF - 1, nch)):
            gds[c] = g_start(c)
        for c in range(nch):
            nxt = c + NBUF - 1
            if nxt < nch:
                if nxt - NBUF >= 0:
                    wds[nxt - NBUF].wait()
                gds[nxt] = g_start(nxt)
            gds[c].wait()
            wds[c] = w_start(c)
            if hook is not None:
                hook(c)
        for c in range(max(0, nch - NBUF), nch):
            wds[c].wait()

    fast = s_start == base

    @pl.when(fast)
    def _():
        out_ds = [None] * NWAVE

        def hook(c):
            if c == 1:
                img_ds[0].wait(); out_ds[0] = img_out(0, base)
            elif c == 2:
                img_ds[1].wait(); out_ds[1] = img_out(1, base)
            elif c == 3:
                img_ds[2].wait(); out_ds[2] = img_out(2, base)
            elif c == 4:
                out_ds[0].wait(); img_ds[3] = img_in(3)
            elif c == 5:
                out_ds[1].wait(); img_ds[4] = img_in(4)
            elif c == 6:
                out_ds[2].wait(); img_ds[5] = img_in(5)
            elif c == 7:
                img_ds[3].wait(); out_ds[3] = img_out(3, base)
            elif c == 8:
                img_ds[4].wait(); out_ds[4] = img_out(4, base)
            elif c == 9:
                img_ds[5].wait(); out_ds[5] = img_out(5, base)
            elif c == 10:
                out_ds[3].wait(); img_ds[6] = img_in(6)
                out_ds[4].wait(); img_ds[7] = img_in(7)
            elif c == 11:
                img_ds[6].wait(); out_ds[6] = img_out(6, base)
                img_ds[7].wait(); out_ds[7] = img_out(7, base)

        run_gather(F, CH, (BLOCK - F) // CH, hook)
        out_ds[5].wait()
        out_ds[6].wait()
        out_ds[7].wait()

    @pl.when(jnp.logical_not(fast))
    def _():
        run_gather(0, CH, BLOCK // CH)
        for k in range(NSLOT):
            img_ds[k].wait()
        prev_out = None
        for k in range(NWAVE):
            if k >= NSLOT:
                d = img_in(k)
                d.wait()
            o = img_out(k, s_start)
            if prev_out is not None:
                prev_out.wait()
            prev_out = o
        prev_out.wait()


@jax.jit
def _run(ids, bounds_flat, table, img_flat):
    mesh = plsc.VectorSubcoreMesh(core_axis_name="c", subcore_axis_name="s")
    k = functools.partial(
        pl.kernel,
        mesh=mesh,
        out_type=jax.ShapeDtypeStruct((SEQ, D), jnp.float32),
        scratch_types=[
            pltpu.VMEM((BLOCK,), jnp.int32),
            pltpu.VMEM((2 * N_SLICES + 16,), jnp.int32),
            pltpu.VMEM((CH, D), jnp.float32),
            pltpu.VMEM((CH, D), jnp.float32),
            pltpu.VMEM_SHARED((NS * NSLOT * IC, D), jnp.float32),
            pltpu.SemaphoreType.DMA,
            pltpu.SemaphoreType.DMA,
            pltpu.SemaphoreType.DMA,
            pltpu.SemaphoreType.DMA,
            pltpu.SemaphoreType.DMA,
            pltpu.SemaphoreType.DMA,
            pltpu.SemaphoreType.DMA,
            pltpu.SemaphoreType.DMA,
            pltpu.SemaphoreType.DMA,
            pltpu.SemaphoreType.DMA,
        ],
    )(_body)
    return k(ids, bounds_flat, table, img_flat)


def kernel(input_ids, image_bounds, embedding_table, image_embeds):
    ids = input_ids.astype(jnp.int32)
    bounds_flat = image_bounds.reshape(-1).astype(jnp.int32)
    img_flat = image_embeds.reshape(-1, image_embeds.shape[-1])
    return _run(ids, bounds_flat, embedding_table, img_flat)
